# baseline (speedup 1.0000x reference)
import functools

import jax
import jax.numpy as jnp
from jax import lax
from jax.experimental import pallas as pl
from jax.experimental.pallas import tpu as pltpu
from jax.experimental.pallas import tpu_sc as plsc

D = 64
DP = 128
L = 16
EPS = 1e-5
NRING = 2


def _rsqrt(a):
    i = plsc.bitcast(a, jnp.int32)
    i = jnp.int32(0x5F3759DF) - lax.shift_right_logical(i, 1)
    y = plsc.bitcast(i, jnp.float32)
    half = a * 0.5
    for _ in range(3):
        y = y * (1.5 - half * y * y)
    return y


@functools.lru_cache(maxsize=None)
def _make_kernel(batch, seq):
    info = plsc.get_sparse_core_info()
    nc, ns = info.num_cores, info.num_subcores
    nw = nc * ns
    bs_per_w = batch // nw
    n4 = bs_per_w // NRING
    assert batch % nw == 0 and bs_per_w % NRING == 0
    nfull = seq // L
    tail = seq - L * nfull
    blocks = nfull + (1 if tail else 0)
    mesh = plsc.VectorSubcoreMesh(core_axis_name="c", subcore_axis_name="s")

    @functools.partial(
        pl.kernel,
        mesh=mesh,
        out_type=jax.ShapeDtypeStruct((batch, seq, D), jnp.float32),
        compiler_params=pltpu.CompilerParams(needs_layout_passes=False),
        scratch_types=[
            [pltpu.VMEM((seq,), jnp.int32) for _ in range(NRING)],
            [pltpu.VMEM((seq,), jnp.int32) for _ in range(NRING)],
            [pltpu.VMEM((seq,), jnp.int32) for _ in range(NRING)],
            [pltpu.VMEM((seq, DP), jnp.float32) for _ in range(NRING)],
            [pltpu.VMEM((seq, D), jnp.float32) for _ in range(NRING)],
            pltpu.VMEM((D,), jnp.float32),
            pltpu.VMEM((D,), jnp.float32),
            pltpu.VMEM((D // 8, DP), jnp.float32),
            pltpu.VMEM((D // 8, DP), jnp.float32),
            [pltpu.SemaphoreType.DMA for _ in range(NRING)],
            [pltpu.SemaphoreType.DMA for _ in range(NRING)],
            [pltpu.SemaphoreType.DMA for _ in range(NRING)],
        ],
    )
    def k(x_hbm, table_hbm, gamma_hbm, beta_hbm, out_hbm,
          raw, sidx, par, rows, obuf, gamma_v, beta_v, dgam, dbet,
          rsem, gsem, wsem):
        wid = lax.axis_index("s") * nc + lax.axis_index("c")
        bb0 = wid * bs_per_w
        pltpu.sync_copy(gamma_hbm, gamma_v)
        pltpu.sync_copy(beta_hbm, beta_v)
        lanes = lax.iota(jnp.int32, L)

        for j0 in range(D):
            cd0 = (lanes + j0) & (D - 1)
            dgam[j0 // 8, pl.ds((j0 % 8) * L, L)] = plsc.load_gather(
                gamma_v, [cd0])
            dbet[j0 // 8, pl.ds((j0 % 8) * L, L)] = plsc.load_gather(
                beta_v, [cd0])

        block_offs = [b * L for b in range(nfull)] + ([seq - L] if tail else [])

        def stage(g, r):
            return pltpu.make_async_copy(x_hbm.at[bb0 + g], raw[r], rsem[r])

        def prep(r):
            for bo in block_offs:
                iv = raw[r][pl.ds(bo, L)]
                sidx[r][pl.ds(bo, L)] = lax.shift_right_logical(iv, 1)
                par[r][pl.ds(bo, L)] = (iv & 1) * D

        def gathers(r):
            return [
                pltpu.make_async_copy(
                    table_hbm.at[sidx[r].at[pl.ds(0, 128)]],
                    rows[r].at[pl.ds(0, 128), :],
                    gsem[r],
                ),
                pltpu.make_async_copy(
                    table_hbm.at[sidx[r].at[pl.ds(128, seq - 128)]],
                    rows[r].at[pl.ds(128, seq - 128), :],
                    gsem[r],
                ),
            ]

        def writeback(g, r):
            return pltpu.make_async_copy(
                obuf[r], out_hbm.at[bb0 + g], wsem[r]
            )

        zero = jnp.zeros((L,), jnp.float32)

        def compute(r):
            rbuf, wbuf = rows[r], obuf[r]

            def block(bo):
                row_idx = bo + lanes
                off = par[r][pl.ds(bo, L)]

                def p1(jo, accs):
                    s0, s1, q0, q1 = accs
                    loaded = []
                    for ji in range(8):
                        cd = ((lanes + (jo * 8 + ji)) & (D - 1)) + off
                        loaded.append(plsc.load_gather(rbuf, [row_idx, cd]))
                    for ji, v in enumerate(loaded):
                        if ji % 2 == 0:
                            s0 = s0 + v
                            q0 = q0 + v * v
                        else:
                            s1 = s1 + v
                            q1 = q1 + v * v
                    return s0, s1, q0, q1

                s0, s1, q0, q1 = lax.fori_loop(
                    0, D // 8, p1, (zero, zero, zero, zero)
                )
                mean = (s0 + s1) * (1.0 / D)
                var = (q0 + q1) * (1.0 / D) - mean * mean
                rstd = _rsqrt(var + EPS)
                mrs = mean * rstd

                def p2(jo, carry3):
                    loaded = []
                    for ji in range(8):
                        j = jo * 8 + ji
                        cd0 = (lanes + j) & (D - 1)
                        v = plsc.load_gather(rbuf, [row_idx, cd0 + off])
                        dg = dgam[jo, pl.ds(ji * L, L)]
                        db = dbet[jo, pl.ds(ji * L, L)]
                        loaded.append((v, cd0, dg, db))
                    for v, cd0, dg, db in loaded:
                        o = (v * rstd - mrs) * dg + db
                        plsc.store_scatter(wbuf, [row_idx, cd0], o)
                    return carry3

                lax.fori_loop(0, D // 8, p2, 0)

            for bo in block_offs:
                block(bo)

        for r in range(NRING):
            stage(r, r).start()
        for r in range(NRING):
            stage(r, r).wait()
            prep(r)
            for gth in gathers(r):
                gth.start()
            if NRING < n4 * NRING:
                stage(r + NRING, r).start()

        def body(i, carry):
            for r in range(NRING):
                g = NRING * i + r
                for gth in gathers(r):
                    gth.wait()

                @pl.when(i > 0)
                def _():
                    writeback(g - NRING, r).wait()

                compute(r)
                writeback(g, r).start()

                @pl.when(i < n4 - 1)
                def _():
                    stage(g + NRING, r).wait()
                    prep(r)
                    for gth in gathers(r):
                        gth.start()

                @pl.when(i < n4 - 2)
                def _():
                    stage(g + 2 * NRING, r).start()
            return carry

        lax.fori_loop(0, n4, body, 0)
        for r in range(NRING):
            writeback(NRING * (n4 - 1) + r, r).wait()

    return k


def kernel(x, table, gamma, beta):
    b, s = x.shape
    v = table.shape[0]
    return _make_kernel(b, s)(x, table.reshape(v // 2, DP), gamma, beta)


# speedup vs baseline: 1.2931x; 1.2931x over previous
import functools

import jax
import jax.numpy as jnp
from jax import lax
from jax.experimental import pallas as pl
from jax.experimental.pallas import tpu as pltpu
from jax.experimental.pallas import tpu_sc as plsc

D = 64
DP = 128
CHUNK = 128
L = 16
EPS = 1e-5
NRING = 4
NOUT = 2


def _rsqrt(a):
    i = plsc.bitcast(a, jnp.int32)
    i = jnp.int32(0x5F3759DF) - lax.shift_right_logical(i, 1)
    y = plsc.bitcast(i, jnp.float32)
    half = a * 0.5
    for _ in range(3):
        y = y * (1.5 - half * y * y)
    return y


@functools.lru_cache(maxsize=None)
def _make_kernel(n_rows):
    info = plsc.get_sparse_core_info()
    nc, ns = info.num_cores, info.num_subcores
    nw = nc * ns
    rows_per_w = n_rows // nw
    n_chunks = rows_per_w // CHUNK
    n4 = n_chunks // NRING
    assert rows_per_w % CHUNK == 0 and n_rows % nw == 0
    assert n_chunks % NRING == 0 and NRING % NOUT == 0
    mesh = plsc.VectorSubcoreMesh(core_axis_name="c", subcore_axis_name="s")

    @functools.partial(
        pl.kernel,
        mesh=mesh,
        out_type=jax.ShapeDtypeStruct((n_rows, D), jnp.float32),
        compiler_params=pltpu.CompilerParams(needs_layout_passes=False),
        scratch_types=[
            pltpu.VMEM((n_chunks, CHUNK), jnp.int32),
            [pltpu.VMEM((CHUNK,), jnp.int32) for _ in range(NRING)],
            [pltpu.VMEM((CHUNK, DP), jnp.float32) for _ in range(NRING)],
            [pltpu.VMEM((CHUNK, D), jnp.float32) for _ in range(NOUT)],
            pltpu.VMEM((D,), jnp.float32),
            pltpu.VMEM((D,), jnp.float32),
            pltpu.VMEM((D // 8, DP), jnp.float32),
            pltpu.VMEM((D // 8, DP), jnp.float32),
            [pltpu.SemaphoreType.DMA for _ in range(NRING)],
            [pltpu.SemaphoreType.DMA for _ in range(NOUT)],
        ],
    )
    def k(x_hbm, table_hbm, gamma_hbm, beta_hbm, out_hbm,
          idx_v, sidx, rows, obuf, gamma_v, beta_v, dgam, dbet, gsem, wsem):
        wid = lax.axis_index("s") * nc + lax.axis_index("c")
        base0 = wid * rows_per_w
        pltpu.sync_copy(gamma_hbm, gamma_v)
        pltpu.sync_copy(beta_hbm, beta_v)
        pltpu.sync_copy(x_hbm.at[pl.ds(wid * n_chunks, n_chunks), :], idx_v)
        lanes = lax.iota(jnp.int32, L)

        for j0 in range(D):
            cd0 = (lanes + j0) & (D - 1)
            dgam[j0 // 8, pl.ds((j0 % 8) * L, L)] = plsc.load_gather(
                gamma_v, [cd0])
            dbet[j0 // 8, pl.ds((j0 % 8) * L, L)] = plsc.load_gather(
                beta_v, [cd0])

        def prep(g, sb):
            for bb in range(CHUNK // L):
                sb[pl.ds(bb * L, L)] = lax.shift_right_logical(
                    idx_v[g, pl.ds(bb * L, L)], 1)

        def gather(g, r):
            return pltpu.make_async_copy(
                table_hbm.at[sidx[r]], rows[r], gsem[r]
            )

        def writeback(g, p):
            return pltpu.make_async_copy(
                obuf[p],
                out_hbm.at[pl.ds(base0 + g * CHUNK, CHUNK)],
                wsem[p],
            )

        zero = jnp.zeros((L,), jnp.float32)

        def compute(g, rbuf, wbuf):
            def block_body(b, carry2):
                row_idx = b * L + lanes
                off = (idx_v[g, pl.ds(b * L, L)] & 1) * D

                def p1(jo, accs):
                    s0, s1, q0, q1 = accs
                    loaded = []
                    for ji in range(8):
                        cd = ((lanes + (jo * 8 + ji)) & (D - 1)) + off
                        loaded.append(plsc.load_gather(rbuf, [row_idx, cd]))
                    for ji, v in enumerate(loaded):
                        if ji % 2 == 0:
                            s0 = s0 + v
                            q0 = q0 + v * v
                        else:
                            s1 = s1 + v
                            q1 = q1 + v * v
                    return s0, s1, q0, q1

                s0, s1, q0, q1 = lax.fori_loop(
                    0, D // 8, p1, (zero, zero, zero, zero)
                )
                mean = (s0 + s1) * (1.0 / D)
                var = (q0 + q1) * (1.0 / D) - mean * mean
                rstd = _rsqrt(var + EPS)
                mrs = mean * rstd

                def p2(jo, carry3):
                    loaded = []
                    for ji in range(8):
                        j = jo * 8 + ji
                        cd0 = (lanes + j) & (D - 1)
                        v = plsc.load_gather(rbuf, [row_idx, cd0 + off])
                        dg = dgam[jo, pl.ds(ji * L, L)]
                        db = dbet[jo, pl.ds(ji * L, L)]
                        loaded.append((v, cd0, dg, db))
                    for v, cd0, dg, db in loaded:
                        o = (v * rstd - mrs) * dg + db
                        plsc.store_scatter(wbuf, [row_idx, cd0], o)
                    return carry3

                lax.fori_loop(0, D // 8, p2, 0)
                return carry2

            lax.fori_loop(0, CHUNK // L, block_body, 0)

        for r in range(NRING):
            prep(r, sidx[r])
            gather(r, r).start()

        def body(i, carry):
            for r in range(NRING):
                g = NRING * i + r
                p = r % NOUT
                gather(g, r).wait()
                if r < NOUT:
                    @pl.when(i > 0)
                    def _():
                        writeback(g - NOUT, p).wait()
                else:
                    writeback(g - NOUT, p).wait()
                compute(g, rows[r], obuf[p])
                writeback(g, p).start()

                @pl.when(i < n4 - 1)
                def _():
                    prep(g + NRING, sidx[r])
                    gather(g + NRING, r).start()
            return carry

        lax.fori_loop(0, n4, body, 0)
        writeback(n_chunks - 2, 0).wait()
        writeback(n_chunks - 1, 1).wait()

    return k


def kernel(x, table, gamma, beta):
    b, s = x.shape
    n = b * s
    v = table.shape[0]
    out = _make_kernel(n)(
        x.reshape(n // CHUNK, CHUNK),
        table.reshape(v // 2, DP),
        gamma,
        beta,
    )
    return out.reshape(b, s, D)
